# baseline (speedup 1.0000x reference)
import functools

import jax
import jax.numpy as jnp
from jax import lax
from jax.experimental import pallas as pl
from jax.experimental.pallas import tpu as pltpu
from jax.experimental.pallas import tpu_sc as plsc

D = 64
DP = 128
NW = 32
G = 128
NBUF = 4
LA = 3
CT = 8192


def _tc_repack(tt):
    V = tt.shape[1]
    grid = (V + CT - 1) // CT

    def repack_body(tt_ref, out_ref):
        out_ref[:, 0:D] = tt_ref[...].T

    return pl.pallas_call(
        repack_body,
        grid=(grid,),
        in_specs=[pl.BlockSpec((D, CT), lambda c: (0, c))],
        out_specs=pl.BlockSpec((CT, DP), lambda c: (c, 0)),
        out_shape=jax.ShapeDtypeStruct((V, DP), jnp.float32),
    )(tt)


@functools.cache
def _make_gather(T, B):
    NBLK = B // G
    assert NBLK == NW and T % NBUF == 0 and T >= 2 * NBUF
    mesh = plsc.VectorSubcoreMesh(core_axis_name="c", subcore_axis_name="s")

    @functools.partial(
        pl.kernel,
        mesh=mesh,
        out_type=jax.ShapeDtypeStruct((T, D // 8, NBLK, 8 * G), jnp.float32),
        scratch_types=[
            pltpu.VMEM((T, G), jnp.int32),
            pltpu.VMEM((NBUF, G, DP), jnp.float32),
            pltpu.VMEM((2, D * G), jnp.float32),
            pltpu.SemaphoreType.DMA((NBUF,)),
            pltpu.SemaphoreType.DMA((2,)),
        ],
        compiler_params=pltpu.CompilerParams(
            use_tc_tiling_on_sc=False, needs_layout_passes=False),
    )
    def gather_k(idx_hbm, table_hbm, out_hbm, idx_v, rows_v, btile_v,
                 gsem, ssem):
        w = lax.axis_index("s") * 2 + lax.axis_index("c")
        pltpu.sync_copy(idx_hbm.at[:, pl.ds(w * G, G)], idx_v)

        def g_start(b, t):
            pltpu.make_async_copy(
                table_hbm.at[idx_v.at[t]], rows_v.at[b], gsem.at[b]).start()

        def g_wait(b):
            pltpu.make_async_copy(
                table_hbm.at[idx_v.at[0]], rows_v.at[b], gsem.at[b]).wait()

        def s_start(p, t):
            for jt in range(D // 8):
                pltpu.make_async_copy(
                    btile_v.at[p, pl.ds(jt * 8 * G, 8 * G)],
                    out_hbm.at[t, jt, w], ssem.at[p]).start()

        def s_wait(p):
            for _ in range(D // 8):
                pltpu.make_async_copy(
                    btile_v.at[p, pl.ds(0, 8 * G)],
                    out_hbm.at[0, 0, w], ssem.at[p]).wait()

        iota = lax.iota(jnp.int32, 16)
        bases = [(iota + 16 * k) * G for k in range(D // 16)]

        def transpose_tile(b, p):
            @plsc.parallel_loop(0, G, unroll=8)
            def _(br):
                for k in range(D // 16):
                    v = rows_v[b, br, pl.ds(16 * k, 16)]
                    plsc.store_scatter(btile_v.at[p], [bases[k] + br], v)

        for b in range(LA):
            g_start(b, b)

        def body(g, carry):
            for v in range(NBUF):
                t = g * NBUF + v
                p = v % 2

                g_wait(v)

                @pl.when(t + LA < T)
                def _():
                    g_start((v + LA) % NBUF, t + LA)

                @pl.when(t >= 2)
                def _():
                    s_wait(p)

                transpose_tile(v, p)
                s_start(p, t)
            return carry

        lax.fori_loop(0, T // NBUF, body, 0)

        for p in range(2):
            s_wait(p)

    return gather_k


def kernel(x, table):
    bsz, hist = x.shape
    xt = x.T
    table_rm = _tc_repack(table.T)
    out4 = _make_gather(hist, bsz)(xt, table_rm)
    out5 = out4.reshape(hist, D // 8, NW, 8, G)
    return out5.transpose(2, 4, 0, 1, 3).reshape(bsz, hist, D)


# speedup vs baseline: 1.9372x; 1.9372x over previous
import functools

import jax
import jax.numpy as jnp
from jax import lax
from jax.experimental import pallas as pl
from jax.experimental.pallas import tpu as pltpu
from jax.experimental.pallas import tpu_sc as plsc

D = 64
DP = 128
NW = 32
G = 128
NBUF = 4
LA = 3
CT = 8192


def _tc_repack(tt):
    V = tt.shape[1]
    grid = (V + CT - 1) // CT

    def repack_body(tt_ref, out_ref):
        out_ref[:, 0:D] = tt_ref[...].T

    return pl.pallas_call(
        repack_body,
        grid=(grid,),
        in_specs=[pl.BlockSpec((D, CT), lambda c: (0, c))],
        out_specs=pl.BlockSpec((CT, DP), lambda c: (c, 0)),
        out_shape=jax.ShapeDtypeStruct((V, DP), jnp.float32),
    )(tt)


@functools.cache
def _make_gather(T, B):
    NBLK = B // G
    assert NBLK == NW and T % NBUF == 0 and T >= 2 * NBUF
    mesh = plsc.VectorSubcoreMesh(core_axis_name="c", subcore_axis_name="s")

    @functools.partial(
        pl.kernel,
        mesh=mesh,
        out_type=jax.ShapeDtypeStruct((T, D // 8, NBLK, 8, G), jnp.float32),
        scratch_types=[
            pltpu.VMEM((T, G), jnp.int32),
            pltpu.VMEM((NBUF, G, DP), jnp.float32),
            pltpu.VMEM((2, D, G + 1), jnp.float32),
            pltpu.SemaphoreType.DMA((NBUF,)),
            pltpu.SemaphoreType.DMA((2,)),
        ],
        compiler_params=pltpu.CompilerParams(
            use_tc_tiling_on_sc=False, needs_layout_passes=False),
    )
    def gather_k(idx_hbm, table_hbm, out_hbm, idx_v, rows_v, btile_v,
                 gsem, ssem):
        w = lax.axis_index("s") * 2 + lax.axis_index("c")
        pltpu.sync_copy(idx_hbm.at[:, pl.ds(w * G, G)], idx_v)

        def g_start(b, t):
            pltpu.make_async_copy(
                table_hbm.at[idx_v.at[t]], rows_v.at[b], gsem.at[b]).start()

        def g_wait(b):
            pltpu.make_async_copy(
                table_hbm.at[idx_v.at[0]], rows_v.at[b], gsem.at[b]).wait()

        def s_start(p, t):
            for jt in range(D // 8):
                pltpu.make_async_copy(
                    btile_v.at[p, pl.ds(jt * 8, 8), pl.ds(0, G)],
                    out_hbm.at[t, jt, w], ssem.at[p]).start()

        def s_wait(p):
            for _ in range(D // 8):
                pltpu.make_async_copy(
                    btile_v.at[p, pl.ds(0, 8), pl.ds(0, G)],
                    out_hbm.at[0, 0, w], ssem.at[p]).wait()

        iota = lax.iota(jnp.int32, 16)
        jrows = [iota + 16 * k for k in range(D // 16)]

        def transpose_tile(b, p):
            @plsc.parallel_loop(0, G, unroll=8)
            def _(br):
                col = jnp.full((16,), br, jnp.int32)
                vs = [rows_v[b, br, pl.ds(16 * k, 16)]
                      for k in range(D // 16)]
                for k in range(D // 16):
                    plsc.store_scatter(btile_v.at[p], [jrows[k], col], vs[k])

        for b in range(LA):
            g_start(b, b)

        def body(g, carry):
            for v in range(NBUF):
                t = g * NBUF + v
                p = v % 2

                g_wait(v)

                @pl.when(t + LA < T)
                def _():
                    g_start((v + LA) % NBUF, t + LA)

                @pl.when(t >= 2)
                def _():
                    s_wait(p)

                transpose_tile(v, p)
                s_start(p, t)
            return carry

        lax.fori_loop(0, T // NBUF, body, 0)

        for p in range(2):
            s_wait(p)

    return gather_k


def kernel(x, table):
    bsz, hist = x.shape
    xt = x.T
    table_rm = _tc_repack(table.T)
    out5 = _make_gather(hist, bsz)(xt, table_rm)
    return out5.transpose(2, 4, 0, 1, 3).reshape(bsz, hist, D)


# speedup vs baseline: 2.0364x; 1.0513x over previous
import functools

import jax
import jax.numpy as jnp
from jax import lax
from jax.experimental import pallas as pl
from jax.experimental.pallas import tpu as pltpu
from jax.experimental.pallas import tpu_sc as plsc

D = 64
NW = 32
G = 128
NBUF = 4
LA = 3
CT = 8192


def _tc_repack(tt):
    V = tt.shape[1]
    grid = (V + CT - 1) // CT

    def repack_body(tt_ref, out_ref):
        t = tt_ref[...].T
        out_ref[:, 0:D] = t[0:CT // 2]
        out_ref[:, D:2 * D] = t[CT // 2:CT]

    return pl.pallas_call(
        repack_body,
        grid=(grid,),
        in_specs=[pl.BlockSpec((D, CT), lambda c: (0, c))],
        out_specs=pl.BlockSpec((CT // 2, 2 * D), lambda c: (c, 0)),
        out_shape=jax.ShapeDtypeStruct((grid * CT // 2, 2 * D), jnp.float32),
    )(tt)


@functools.cache
def _make_gather(T, B):
    NBLK = B // G
    assert NBLK == NW and T % NBUF == 0 and T >= 2 * NBUF
    mesh = plsc.VectorSubcoreMesh(core_axis_name="c", subcore_axis_name="s")

    @functools.partial(
        pl.kernel,
        mesh=mesh,
        out_type=jax.ShapeDtypeStruct((T, D // 8, NBLK, 8, G), jnp.float32),
        scratch_types=[
            pltpu.VMEM((T, G), jnp.int32),
            pltpu.VMEM((NBUF, G, D), jnp.float32),
            pltpu.VMEM((2, D, G + 1), jnp.float32),
            pltpu.SemaphoreType.DMA((NBUF,)),
            pltpu.SemaphoreType.DMA((2,)),
        ],
        compiler_params=pltpu.CompilerParams(
            use_tc_tiling_on_sc=False, needs_layout_passes=False),
    )
    def gather_k(idx_hbm, table_hbm, out_hbm, idx_v, rows_v, btile_v,
                 gsem, ssem):
        w = lax.axis_index("s") * 2 + lax.axis_index("c")
        pltpu.sync_copy(idx_hbm.at[:, pl.ds(w * G, G)], idx_v)

        @plsc.parallel_loop(0, T, unroll=4)
        def _(t):
            for m in range(G // 16):
                i = idx_v[t, pl.ds(16 * m, 16)]
                idx_v[t, pl.ds(16 * m, 16)] = (
                    (i & ~jnp.int32(CT - 1))
                    + ((i & jnp.int32(CT // 2 - 1)) << 1)
                    + ((i >> (CT // 2).bit_length() - 1) & 1))

        def g_start(b, t):
            pltpu.make_async_copy(
                table_hbm.at[idx_v.at[t]], rows_v.at[b], gsem.at[b]).start()

        def g_wait(b):
            pltpu.make_async_copy(
                table_hbm.at[idx_v.at[0]], rows_v.at[b], gsem.at[b]).wait()

        def s_start(p, t):
            for jt in range(D // 8):
                pltpu.make_async_copy(
                    btile_v.at[p, pl.ds(jt * 8, 8), pl.ds(0, G)],
                    out_hbm.at[t, jt, w], ssem.at[p]).start()

        def s_wait(p):
            for _ in range(D // 8):
                pltpu.make_async_copy(
                    btile_v.at[p, pl.ds(0, 8), pl.ds(0, G)],
                    out_hbm.at[0, 0, w], ssem.at[p]).wait()

        iota = lax.iota(jnp.int32, 16)
        jrows = [iota + 16 * k for k in range(D // 16)]

        def transpose_tile(b, p):
            @plsc.parallel_loop(0, G, unroll=8)
            def _(br):
                col = jnp.full((16,), br, jnp.int32)
                vs = [rows_v[b, br, pl.ds(16 * k, 16)]
                      for k in range(D // 16)]
                for k in range(D // 16):
                    plsc.store_scatter(btile_v.at[p], [jrows[k], col], vs[k])

        for b in range(LA):
            g_start(b, b)

        def body(g, carry):
            for v in range(NBUF):
                t = g * NBUF + v
                p = v % 2

                g_wait(v)

                @pl.when(t + LA < T)
                def _():
                    g_start((v + LA) % NBUF, t + LA)

                @pl.when(t >= 2)
                def _():
                    s_wait(p)

                transpose_tile(v, p)
                s_start(p, t)
            return carry

        lax.fori_loop(0, T // NBUF, body, 0)

        for p in range(2):
            s_wait(p)

    return gather_k


def kernel(x, table):
    bsz, hist = x.shape
    xt = x.T
    table_rm = _tc_repack(table.T).reshape(-1, D)
    out5 = _make_gather(hist, bsz)(xt, table_rm)
    return out5.transpose(2, 4, 0, 1, 3).reshape(bsz, hist, D)


# speedup vs baseline: 2.1827x; 1.0718x over previous
import functools

import jax
import jax.numpy as jnp
from jax import lax
from jax.experimental import pallas as pl
from jax.experimental.pallas import tpu as pltpu
from jax.experimental.pallas import tpu_sc as plsc

D = 64
NW = 32
G = 128
NBUF = 4
LA = 3
CT = 16384


def _tc_repack(tt):
    V = tt.shape[1]
    grid = (V + CT - 1) // CT

    def repack_body(tt_ref, out_ref):
        t = tt_ref[...].T
        out_ref[:, 0:D] = t[0:CT // 2]
        out_ref[:, D:2 * D] = t[CT // 2:CT]

    return pl.pallas_call(
        repack_body,
        grid=(grid,),
        in_specs=[pl.BlockSpec((D, CT), lambda c: (0, c))],
        out_specs=pl.BlockSpec((CT // 2, 2 * D), lambda c: (c, 0)),
        out_shape=jax.ShapeDtypeStruct((grid * CT // 2, 2 * D), jnp.float32),
    )(tt)


@functools.cache
def _make_gather(T, B):
    NBLK = B // G
    assert NBLK == NW and T % NBUF == 0 and T >= 2 * NBUF
    mesh = plsc.VectorSubcoreMesh(core_axis_name="c", subcore_axis_name="s")

    @functools.partial(
        pl.kernel,
        mesh=mesh,
        out_type=jax.ShapeDtypeStruct((T, D // 8, NBLK, 8, G), jnp.float32),
        scratch_types=[
            pltpu.VMEM((T, G), jnp.int32),
            pltpu.VMEM((NBUF, G, D), jnp.float32),
            pltpu.VMEM((2, D, G + 1), jnp.float32),
            pltpu.SemaphoreType.DMA((NBUF,)),
            pltpu.SemaphoreType.DMA((2,)),
        ],
        compiler_params=pltpu.CompilerParams(
            use_tc_tiling_on_sc=False, needs_layout_passes=False),
    )
    def gather_k(idx_hbm, table_hbm, out_hbm, idx_v, rows_v, btile_v,
                 gsem, ssem):
        w = lax.axis_index("s") * 2 + lax.axis_index("c")
        pltpu.sync_copy(idx_hbm.at[:, pl.ds(w * G, G)], idx_v)

        @plsc.parallel_loop(0, T, unroll=4)
        def _(t):
            for m in range(G // 16):
                i = idx_v[t, pl.ds(16 * m, 16)]
                idx_v[t, pl.ds(16 * m, 16)] = (
                    (i & ~jnp.int32(CT - 1))
                    + ((i & jnp.int32(CT // 2 - 1)) << 1)
                    + ((i >> (CT // 2).bit_length() - 1) & 1))

        def g_start(b, t):
            pltpu.make_async_copy(
                table_hbm.at[idx_v.at[t]], rows_v.at[b], gsem.at[b]).start()

        def g_wait(b):
            pltpu.make_async_copy(
                table_hbm.at[idx_v.at[0]], rows_v.at[b], gsem.at[b]).wait()

        def s_start(p, t):
            for jt in range(D // 8):
                pltpu.make_async_copy(
                    btile_v.at[p, pl.ds(jt * 8, 8), pl.ds(0, G)],
                    out_hbm.at[t, jt, w], ssem.at[p]).start()

        def s_wait(p):
            for _ in range(D // 8):
                pltpu.make_async_copy(
                    btile_v.at[p, pl.ds(0, 8), pl.ds(0, G)],
                    out_hbm.at[0, 0, w], ssem.at[p]).wait()

        iota = lax.iota(jnp.int32, 16)
        jrows = [iota + 16 * k for k in range(D // 16)]

        def transpose_tile(b, p):
            @plsc.parallel_loop(0, G, unroll=8)
            def _(br):
                col = jnp.full((16,), br, jnp.int32)
                vs = [rows_v[b, br, pl.ds(16 * k, 16)]
                      for k in range(D // 16)]
                for k in range(D // 16):
                    plsc.store_scatter(btile_v.at[p], [jrows[k], col], vs[k])

        for b in range(LA):
            g_start(b, b)

        def body(g, carry):
            for v in range(NBUF):
                t = g * NBUF + v
                p = v % 2

                g_wait(v)

                @pl.when(t + LA < T)
                def _():
                    g_start((v + LA) % NBUF, t + LA)

                @pl.when(t >= 2)
                def _():
                    s_wait(p)

                transpose_tile(v, p)
                s_start(p, t)
            return carry

        lax.fori_loop(0, T // NBUF, body, 0)

        for p in range(2):
            s_wait(p)

    return gather_k


def kernel(x, table):
    bsz, hist = x.shape
    xt = x.T
    table_rm = _tc_repack(table.T).reshape(-1, D)
    out5 = _make_gather(hist, bsz)(xt, table_rm)
    return out5.transpose(2, 4, 0, 1, 3).reshape(bsz, hist, D)


# speedup vs baseline: 2.2518x; 1.0316x over previous
import functools

import jax
import jax.numpy as jnp
from jax import lax
from jax.experimental import pallas as pl
from jax.experimental.pallas import tpu as pltpu
from jax.experimental.pallas import tpu_sc as plsc

D = 64
NW = 32
G = 128
NBUF = 4
LA = 3
CT = 32768


def _tc_repack(tt):
    V = tt.shape[1]
    grid = (V + CT - 1) // CT

    def repack_body(tt_ref, out_ref):
        t = tt_ref[...].T
        out_ref[:, 0:D] = t[0:CT // 2]
        out_ref[:, D:2 * D] = t[CT // 2:CT]

    return pl.pallas_call(
        repack_body,
        grid=(grid,),
        in_specs=[pl.BlockSpec((D, CT), lambda c: (0, c))],
        out_specs=pl.BlockSpec((CT // 2, 2 * D), lambda c: (c, 0)),
        out_shape=jax.ShapeDtypeStruct((grid * CT // 2, 2 * D), jnp.float32),
    )(tt)


@functools.cache
def _make_gather(T, B):
    NBLK = B // G
    assert NBLK == NW and T % NBUF == 0 and T >= 2 * NBUF
    mesh = plsc.VectorSubcoreMesh(core_axis_name="c", subcore_axis_name="s")

    @functools.partial(
        pl.kernel,
        mesh=mesh,
        out_type=jax.ShapeDtypeStruct((T, D // 8, NBLK, 8, G), jnp.float32),
        scratch_types=[
            pltpu.VMEM((T, G), jnp.int32),
            pltpu.VMEM((NBUF, G, D), jnp.float32),
            pltpu.VMEM((2, D, G + 1), jnp.float32),
            pltpu.SemaphoreType.DMA((NBUF,)),
            pltpu.SemaphoreType.DMA((2,)),
        ],
        compiler_params=pltpu.CompilerParams(
            use_tc_tiling_on_sc=False, needs_layout_passes=False),
    )
    def gather_k(idx_hbm, table_hbm, out_hbm, idx_v, rows_v, btile_v,
                 gsem, ssem):
        w = lax.axis_index("s") * 2 + lax.axis_index("c")
        pltpu.sync_copy(idx_hbm.at[:, pl.ds(w * G, G)], idx_v)

        @plsc.parallel_loop(0, T, unroll=4)
        def _(t):
            for m in range(G // 16):
                i = idx_v[t, pl.ds(16 * m, 16)]
                idx_v[t, pl.ds(16 * m, 16)] = (
                    (i & ~jnp.int32(CT - 1))
                    + ((i & jnp.int32(CT // 2 - 1)) << 1)
                    + ((i >> (CT // 2).bit_length() - 1) & 1))

        def g_start(b, t):
            pltpu.make_async_copy(
                table_hbm.at[idx_v.at[t]], rows_v.at[b], gsem.at[b]).start()

        def g_wait(b):
            pltpu.make_async_copy(
                table_hbm.at[idx_v.at[0]], rows_v.at[b], gsem.at[b]).wait()

        def s_start(p, t):
            for jt in range(D // 8):
                pltpu.make_async_copy(
                    btile_v.at[p, pl.ds(jt * 8, 8), pl.ds(0, G)],
                    out_hbm.at[t, jt, w], ssem.at[p]).start()

        def s_wait(p):
            for _ in range(D // 8):
                pltpu.make_async_copy(
                    btile_v.at[p, pl.ds(0, 8), pl.ds(0, G)],
                    out_hbm.at[0, 0, w], ssem.at[p]).wait()

        iota = lax.iota(jnp.int32, 16)
        jrows = [iota + 16 * k for k in range(D // 16)]

        def transpose_tile(b, p):
            @plsc.parallel_loop(0, G, unroll=8)
            def _(br):
                col = jnp.full((16,), br, jnp.int32)
                vs = [rows_v[b, br, pl.ds(16 * k, 16)]
                      for k in range(D // 16)]
                for k in range(D // 16):
                    plsc.store_scatter(btile_v.at[p], [jrows[k], col], vs[k])

        for b in range(LA):
            g_start(b, b)

        def body(g, carry):
            for v in range(NBUF):
                t = g * NBUF + v
                p = v % 2

                g_wait(v)

                @pl.when(t + LA < T)
                def _():
                    g_start((v + LA) % NBUF, t + LA)

                @pl.when(t >= 2)
                def _():
                    s_wait(p)

                transpose_tile(v, p)
                s_start(p, t)
            return carry

        lax.fori_loop(0, T // NBUF, body, 0)

        for p in range(2):
            s_wait(p)

    return gather_k


def kernel(x, table):
    bsz, hist = x.shape
    xt = x.T
    table_rm = _tc_repack(table.T).reshape(-1, D)
    out5 = _make_gather(hist, bsz)(xt, table_rm)
    return out5.transpose(2, 4, 0, 1, 3).reshape(bsz, hist, D)
